# baseline (speedup 1.0000x reference)
import functools

import jax
import jax.numpy as jnp
from jax import lax
from jax.experimental import pallas as pl
from jax.experimental.pallas import tpu as pltpu
from jax.experimental.pallas import tpu_sc as plsc

PERCENTILE = 0.999
N = 2 * 4096 * 2048
ROWS, COLS = 8192, 2048
NW = 32
ROWS_W = ROWS // NW
CHUNK_R = 8
NCHUNKS = ROWS_W // CHUNK_R
NBUF = 4
SHIFT = 17
NBINS = 1 << (32 - SHIFT)
L = 16

_H_MAX = PERCENTILE * (N - 1)
_H_MIN = (1.0 - PERCENTILE) * (N - 1)
RANK_MAX = int(_H_MAX)
FRAC_MAX = _H_MAX - RANK_MAX
RANK_MIN = int(_H_MIN)
FRAC_MIN = _H_MIN - RANK_MIN


def _sc_histogram(x_i32):
  mesh = plsc.VectorSubcoreMesh(core_axis_name="c", subcore_axis_name="s")

  @functools.partial(
      pl.kernel,
      out_type=(
          jax.ShapeDtypeStruct((NW, NBINS // 128, 128), jnp.int32),
          jax.ShapeDtypeStruct((ROWS, COLS), jnp.int32),
      ),
      mesh=mesh,
      scratch_types=[
          pltpu.VMEM((NBINS // 128, 128), jnp.int32),
          pltpu.VMEM((CHUNK_R, COLS), jnp.int32),
          pltpu.VMEM((CHUNK_R, COLS), jnp.int32),
          pltpu.VMEM((CHUNK_R, COLS), jnp.int32),
          pltpu.VMEM((CHUNK_R, COLS), jnp.int32),
          pltpu.SemaphoreType.DMA,
          pltpu.SemaphoreType.DMA,
          pltpu.SemaphoreType.DMA,
          pltpu.SemaphoreType.DMA,
          pltpu.SemaphoreType.DMA,
          pltpu.SemaphoreType.DMA,
          pltpu.SemaphoreType.DMA,
          pltpu.SemaphoreType.DMA,
      ],
      compiler_params=pltpu.CompilerParams(
          needs_layout_passes=False, use_tc_tiling_on_sc=True),
  )
  def hist_kernel(x_hbm, out_hbm, xout_hbm, hist,
                  buf0, buf1, buf2, buf3,
                  isem0, isem1, isem2, isem3,
                  osem0, osem1, osem2, osem3):
    nc = 2
    wid = lax.axis_index("s") * nc + lax.axis_index("c")
    row0 = wid * ROWS_W
    bufs = (buf0, buf1, buf2, buf3)
    isems = (isem0, isem1, isem2, isem3)
    osems = (osem0, osem1, osem2, osem3)

    zeros = jnp.zeros((L,), jnp.int32)

    def zero_body(i, carry):
      hist[i, pl.ds(0, L)] = zeros
      hist[i, pl.ds(L, L)] = zeros
      hist[i, pl.ds(2 * L, L)] = zeros
      hist[i, pl.ds(3 * L, L)] = zeros
      hist[i, pl.ds(4 * L, L)] = zeros
      hist[i, pl.ds(5 * L, L)] = zeros
      hist[i, pl.ds(6 * L, L)] = zeros
      hist[i, pl.ds(7 * L, L)] = zeros
      return carry

    lax.fori_loop(0, NBINS // 128, zero_body, 0, unroll=4)

    ones = jnp.full((L,), 1, jnp.int32)
    ilv = 16

    def bin_body(buf):
      for r in range(CHUNK_R):
        def vec_body(j, carry):
          base_e = j * (L * ilv)
          us = [buf[r, pl.ds(base_e + k * L, L)] for k in range(ilv)]
          bs = [lax.shift_right_logical(u, SHIFT) for u in us]
          rcs = [(lax.shift_right_logical(b, 7),
                  lax.bitwise_and(b, jnp.int32(127))) for b in bs]
          for br, bc in rcs:
            plsc.addupdate_scatter(hist, [br, bc], ones)
          return carry

        lax.fori_loop(0, COLS // (L * ilv), vec_body, 0, unroll=1)

    def in_copy(c, b):
      return pltpu.make_async_copy(
          x_hbm.at[pl.ds(row0 + c * CHUNK_R, CHUNK_R)], bufs[b], isems[b])

    def out_copy(c, b):
      return pltpu.make_async_copy(
          bufs[b], xout_hbm.at[pl.ds(row0 + c * CHUNK_R, CHUNK_R)], osems[b])

    in_copy(0, 0).start()
    in_copy(1, 1).start()
    in_copy(2, 2).start()

    def round_body(r, carry):
      c0 = r * NBUF
      for i in range(NBUF):
        c = c0 + i
        in_copy(c, i).wait()
        bin_body(bufs[i])
        out_copy(c, i).start()
        j = (i + 3) % NBUF
        if i == 0:
          @pl.when(r > 0)
          def _():
            out_copy(c - 1, j).wait()
          in_copy(c + 3, j).start()
        else:
          @pl.when(r < NCHUNKS // NBUF - 1)
          def _():
            out_copy(c - 1, j).wait()
            in_copy(c + 3, j).start()
      return carry

    lax.fori_loop(0, NCHUNKS // NBUF, round_body, 0)

    for i in range(NBUF):
      out_copy(NCHUNKS - NBUF + i, i).wait()

    pltpu.sync_copy(hist, out_hbm.at[wid])

  return hist_kernel(x_i32)


def _bwd_cumsum_lanes(h):
  k = 1
  while k < h.shape[1]:
    shifted = jnp.concatenate(
        [h[:, k:], jnp.zeros((h.shape[0], k), jnp.int32)], axis=1)
    h = h + shifted
    k *= 2
  return h


def _bwd_cumsum_rows(h):
  k = 1
  while k < h.shape[0]:
    shifted = jnp.concatenate(
        [h[k:, :], jnp.zeros((k, h.shape[1]), jnp.int32)], axis=0)
    h = h + shifted
    k *= 2
  return h


def _decode_key(key):
  u = jnp.where(key < 0, key & jnp.int32(0x7FFFFFFF), ~key)
  return lax.bitcast_convert_type(u, jnp.float32)


def _tc_select(hist3):
  rows = NBINS // 128

  def select_kernel(h_ref, min_ref, max_ref):
    h = jnp.sum(h_ref[...], axis=0)
    bwd_lane = _bwd_cumsum_lanes(h)
    row_tot = bwd_lane[:, :1]
    row_suf_excl = _bwd_cumsum_rows(row_tot) - row_tot
    bwd_inc = row_suf_excl + bwd_lane

    r_idx = lax.broadcasted_iota(jnp.int32, (rows, 128), 0)
    c_idx = lax.broadcasted_iota(jnp.int32, (rows, 128), 1)
    raw = r_idx * 128 + c_idx
    isneg = raw >= NBINS // 2
    negtotal = bwd_inc[(NBINS // 2) // 128:(NBINS // 2) // 128 + 1, 0:1]
    cinc = jnp.where(isneg, bwd_inc, N - bwd_inc + h + negtotal)
    cexc = cinc - h
    jgrid = jnp.where(isneg, (NBINS - 1) - raw, raw + NBINS // 2)

    def quantile_at(rank, frac):
      rank = jnp.int32(rank)
      mask = (cexc <= rank) & (rank < cinc)
      b = jnp.max(jnp.where(mask, jgrid, 0))
      cnt = jnp.max(jnp.where(mask, h, 0))
      ce = jnp.max(jnp.where(mask, cexc, 0))
      lo = _decode_key(b << SHIFT)
      hi = _decode_key((b + 1) << SHIFT)
      t = ((rank - ce).astype(jnp.float32) + jnp.float32(frac + 0.5)) / (
          cnt.astype(jnp.float32))
      t = jnp.clip(t, 0.0, 1.0)
      return lo + t * (hi - lo)

    min_ref[...] = jnp.broadcast_to(quantile_at(RANK_MIN, FRAC_MIN), (1, 1))
    max_ref[...] = jnp.broadcast_to(quantile_at(RANK_MAX, FRAC_MAX), (1, 1))

  return pl.pallas_call(
      select_kernel,
      out_shape=(
          jax.ShapeDtypeStruct((1, 1), jnp.float32),
          jax.ShapeDtypeStruct((1, 1), jnp.float32),
      ),
  )(hist3)


def kernel(x):
  x_i32 = lax.bitcast_convert_type(x, jnp.int32).reshape(ROWS, COLS)
  hist_raw, xout_i32 = _sc_histogram(x_i32)
  minv, maxv = _tc_select(hist_raw)
  xc = lax.bitcast_convert_type(xout_i32, jnp.float32).reshape(x.shape)
  return (xc, minv.reshape(()), maxv.reshape(()))


# speedup vs baseline: 1.9863x; 1.9863x over previous
import functools

import jax
import jax.numpy as jnp
from jax import lax
from jax.experimental import pallas as pl
from jax.experimental.pallas import tpu as pltpu
from jax.experimental.pallas import tpu_sc as plsc

PERCENTILE = 0.999
N = 2 * 4096 * 2048
ROWS, COLS = 8192, 2048
NW = 32
ROWS_W = ROWS // NW
CHUNK_R = 8
NCHUNKS = ROWS_W // CHUNK_R
NBUF = 4
SHIFT = 17
NBINS = 1 << (32 - SHIFT)
L = 16

_H_MAX = PERCENTILE * (N - 1)
_H_MIN = (1.0 - PERCENTILE) * (N - 1)
RANK_MAX = int(_H_MAX)
FRAC_MAX = _H_MAX - RANK_MAX
RANK_MIN = int(_H_MIN)
FRAC_MIN = _H_MIN - RANK_MIN


def _sc_histogram(x):
  mesh = plsc.VectorSubcoreMesh(core_axis_name="c", subcore_axis_name="s")

  @functools.partial(
      pl.kernel,
      out_type=(
          jax.ShapeDtypeStruct((NW, NBINS // 128, 128), jnp.int32),
          jax.ShapeDtypeStruct(x.shape, jnp.float32),
      ),
      mesh=mesh,
      scratch_types=[
          pltpu.VMEM((NBINS // 128, 128), jnp.int32),
          pltpu.VMEM((CHUNK_R, COLS), jnp.float32),
          pltpu.VMEM((CHUNK_R, COLS), jnp.float32),
          pltpu.VMEM((CHUNK_R, COLS), jnp.float32),
          pltpu.VMEM((CHUNK_R, COLS), jnp.float32),
          pltpu.SemaphoreType.DMA,
          pltpu.SemaphoreType.DMA,
          pltpu.SemaphoreType.DMA,
          pltpu.SemaphoreType.DMA,
          pltpu.SemaphoreType.DMA,
          pltpu.SemaphoreType.DMA,
          pltpu.SemaphoreType.DMA,
          pltpu.SemaphoreType.DMA,
      ],
      compiler_params=pltpu.CompilerParams(
          needs_layout_passes=False, use_tc_tiling_on_sc=True),
  )
  def hist_kernel(x_hbm, out_hbm, xout_hbm, hist,
                  buf0, buf1, buf2, buf3,
                  isem0, isem1, isem2, isem3,
                  osem0, osem1, osem2, osem3):
    nc = 2
    wid = lax.axis_index("s") * nc + lax.axis_index("c")
    row0 = wid * ROWS_W
    d0 = lax.shift_right_logical(row0, 12)
    r1 = lax.bitwise_and(row0, 4095)
    bufs = (buf0, buf1, buf2, buf3)
    isems = (isem0, isem1, isem2, isem3)
    osems = (osem0, osem1, osem2, osem3)

    zeros = jnp.zeros((L,), jnp.int32)

    def zero_body(i, carry):
      hist[i, pl.ds(0, L)] = zeros
      hist[i, pl.ds(L, L)] = zeros
      hist[i, pl.ds(2 * L, L)] = zeros
      hist[i, pl.ds(3 * L, L)] = zeros
      hist[i, pl.ds(4 * L, L)] = zeros
      hist[i, pl.ds(5 * L, L)] = zeros
      hist[i, pl.ds(6 * L, L)] = zeros
      hist[i, pl.ds(7 * L, L)] = zeros
      return carry

    lax.fori_loop(0, NBINS // 128, zero_body, 0, unroll=4)

    ones = jnp.full((L,), 1, jnp.int32)
    ilv = 16

    def bin_body(buf):
      for r in range(CHUNK_R):
        def vec_body(j, carry):
          base_e = j * (L * ilv)
          us = [plsc.bitcast(buf[r, pl.ds(base_e + k * L, L)], jnp.int32)
                for k in range(ilv)]
          bs = [lax.shift_right_logical(u, SHIFT) for u in us]
          rcs = [(lax.shift_right_logical(b, 7),
                  lax.bitwise_and(b, jnp.int32(127))) for b in bs]
          for br, bc in rcs:
            plsc.addupdate_scatter(hist, [br, bc], ones)
          return carry

        lax.fori_loop(0, COLS // (L * ilv), vec_body, 0, unroll=1)

    def in_copy(c, b):
      off = pl.multiple_of(r1 + c * CHUNK_R, CHUNK_R)
      return pltpu.make_async_copy(
          x_hbm.at[d0, pl.ds(off, CHUNK_R)], bufs[b], isems[b])

    def out_copy(c, b):
      off = pl.multiple_of(r1 + c * CHUNK_R, CHUNK_R)
      return pltpu.make_async_copy(
          bufs[b], xout_hbm.at[d0, pl.ds(off, CHUNK_R)], osems[b])

    in_copy(0, 0).start()
    in_copy(1, 1).start()
    in_copy(2, 2).start()

    def round_body(r, carry):
      c0 = r * NBUF
      for i in range(NBUF):
        c = c0 + i
        in_copy(c, i).wait()
        bin_body(bufs[i])
        out_copy(c, i).start()
        j = (i + 3) % NBUF
        if i == 0:
          @pl.when(r > 0)
          def _():
            out_copy(c - 1, j).wait()
          in_copy(c + 3, j).start()
        else:
          @pl.when(r < NCHUNKS // NBUF - 1)
          def _():
            out_copy(c - 1, j).wait()
            in_copy(c + 3, j).start()
      return carry

    lax.fori_loop(0, NCHUNKS // NBUF, round_body, 0)

    for i in range(NBUF):
      out_copy(NCHUNKS - NBUF + i, i).wait()

    pltpu.sync_copy(hist, out_hbm.at[wid])

  return hist_kernel(x)


def _bwd_cumsum_lanes(h):
  k = 1
  while k < h.shape[1]:
    shifted = jnp.concatenate(
        [h[:, k:], jnp.zeros((h.shape[0], k), jnp.int32)], axis=1)
    h = h + shifted
    k *= 2
  return h


def _bwd_cumsum_rows(h):
  k = 1
  while k < h.shape[0]:
    shifted = jnp.concatenate(
        [h[k:, :], jnp.zeros((k, h.shape[1]), jnp.int32)], axis=0)
    h = h + shifted
    k *= 2
  return h


def _decode_key(key):
  u = jnp.where(key < 0, key & jnp.int32(0x7FFFFFFF), ~key)
  return lax.bitcast_convert_type(u, jnp.float32)


def _tc_select(hist3):
  rows = NBINS // 128

  def select_kernel(h_ref, min_ref, max_ref):
    h = jnp.sum(h_ref[...], axis=0)
    bwd_lane = _bwd_cumsum_lanes(h)
    row_tot = bwd_lane[:, :1]
    row_suf_excl = _bwd_cumsum_rows(row_tot) - row_tot
    bwd_inc = row_suf_excl + bwd_lane

    r_idx = lax.broadcasted_iota(jnp.int32, (rows, 128), 0)
    c_idx = lax.broadcasted_iota(jnp.int32, (rows, 128), 1)
    raw = r_idx * 128 + c_idx
    isneg = raw >= NBINS // 2
    negtotal = bwd_inc[(NBINS // 2) // 128:(NBINS // 2) // 128 + 1, 0:1]
    cinc = jnp.where(isneg, bwd_inc, N - bwd_inc + h + negtotal)
    cexc = cinc - h
    jgrid = jnp.where(isneg, (NBINS - 1) - raw, raw + NBINS // 2)

    def quantile_at(rank, frac):
      rank = jnp.int32(rank)
      mask = (cexc <= rank) & (rank < cinc)
      b = jnp.max(jnp.where(mask, jgrid, 0))
      cnt = jnp.max(jnp.where(mask, h, 0))
      ce = jnp.max(jnp.where(mask, cexc, 0))
      lo = _decode_key(b << SHIFT)
      hi = _decode_key((b + 1) << SHIFT)
      t = ((rank - ce).astype(jnp.float32) + jnp.float32(frac + 0.5)) / (
          cnt.astype(jnp.float32))
      t = jnp.clip(t, 0.0, 1.0)
      return lo + t * (hi - lo)

    min_ref[...] = jnp.broadcast_to(quantile_at(RANK_MIN, FRAC_MIN), (1, 1))
    max_ref[...] = jnp.broadcast_to(quantile_at(RANK_MAX, FRAC_MAX), (1, 1))

  return pl.pallas_call(
      select_kernel,
      out_shape=(
          jax.ShapeDtypeStruct((1, 1), jnp.float32),
          jax.ShapeDtypeStruct((1, 1), jnp.float32),
      ),
  )(hist3)


def kernel(x):
  hist_raw, xc = _sc_histogram(x)
  minv, maxv = _tc_select(hist_raw)
  return (xc, minv.reshape(()), maxv.reshape(()))
